# baseline (speedup 1.0000x reference)
import functools

import jax
import jax.numpy as jnp
from jax import lax
from jax.experimental import pallas as pl
from jax.experimental.pallas import tpu as pltpu
from jax.experimental.pallas import tpu_sc as plsc

B, L = 4096, 200
D_A, D_C = 64, 32
D_OUT = D_A + D_C
BL = B * L
PA, PC = D_A // 2, D_C // 2
PO = PA + PC

NC, NS = 2, 16
NW = NC * NS
CH = 128
PER_W = BL // NW
NITER = PER_W // CH
NBUF = 8
PF = 5
HEAD = 8
TAIL = 8
assert (NITER - HEAD - TAIL) % NBUF == 0 and PF < NBUF <= HEAD + (NBUF - PF)


def _emb_body(aidx_hbm, cidx_hbm, emb_hbm, chem_hbm, out_hbm,
              aidx_v, cidx_v, abuf, cbuf, gsems, wsems):
    wid = lax.axis_index("s") * NC + lax.axis_index("c")
    row0 = wid * PER_W
    it0 = wid * NITER

    pltpu.sync_copy(aidx_hbm.at[pl.ds(it0, NITER)], aidx_v)
    pltpu.sync_copy(cidx_hbm.at[pl.ds(it0, NITER)], cidx_v)

    def gather_start(j, b):
        pltpu.async_copy(emb_hbm.at[aidx_v.at[j]], abuf.at[b], gsems.at[b])
        pltpu.async_copy(chem_hbm.at[cidx_v.at[j]], cbuf.at[b], gsems.at[b])

    def gather_wait(b):
        pltpu.make_async_copy(emb_hbm.at[aidx_v.at[0]], abuf.at[b],
                              gsems.at[b]).wait()
        pltpu.make_async_copy(chem_hbm.at[cidx_v.at[0]], cbuf.at[b],
                              gsems.at[b]).wait()

    def write_start(j, b):
        r = row0 + j * CH
        pltpu.async_copy(abuf.at[b], out_hbm.at[pl.ds(r, CH), pl.ds(0, PA)],
                         wsems.at[b])
        pltpu.async_copy(cbuf.at[b], out_hbm.at[pl.ds(r, CH), pl.ds(PA, PC)],
                         wsems.at[b])

    def write_wait(b):
        pltpu.make_async_copy(abuf.at[b],
                              out_hbm.at[pl.ds(row0, CH), pl.ds(0, PA)],
                              wsems.at[b]).wait()
        pltpu.make_async_copy(cbuf.at[b],
                              out_hbm.at[pl.ds(row0, CH), pl.ds(PA, PC)],
                              wsems.at[b]).wait()

    def step(j, b, bn, wait_w, prefetch):
        gather_wait(b)
        write_start(j, b)
        if prefetch:
            if wait_w:
                write_wait(bn)
            gather_start(j + PF, bn)

    for p in range(PF):
        gather_start(p, p % NBUF)

    for j in range(HEAD):
        step(j, j % NBUF, (j + PF) % NBUF, wait_w=(j >= NBUF - PF),
             prefetch=True)

    @pl.loop(HEAD, NITER - TAIL, step=NBUF)
    def _main(g):
        for b in range(NBUF):
            step(g + b, b, (b + PF) % NBUF, wait_w=True, prefetch=True)

    for j in range(NITER - TAIL, NITER):
        step(j, j % NBUF, (j + PF) % NBUF, wait_w=True,
             prefetch=(j + PF < NITER))

    for w in range(NITER - NBUF, NITER):
        write_wait(w % NBUF)


_emb_lookup = functools.partial(
    pl.kernel,
    out_type=jax.ShapeDtypeStruct((BL, 128), jnp.float32),
    mesh=plsc.VectorSubcoreMesh(core_axis_name="c", subcore_axis_name="s",
                                num_cores=NC, num_subcores=NS),
    scratch_types=[
        pltpu.VMEM((NITER, CH), jnp.int32),
        pltpu.VMEM((NITER, CH), jnp.int32),
        pltpu.VMEM((NBUF, CH, PA), jnp.float32),
        pltpu.VMEM((NBUF, CH, PC), jnp.float32),
        pltpu.SemaphoreType.DMA((NBUF,)),
        pltpu.SemaphoreType.DMA((NBUF,)),
    ],
    compiler_params=pltpu.CompilerParams(use_tc_tiling_on_sc=False),
)(_emb_body)


def _pack_table(table):
    t16 = table.astype(jnp.bfloat16).reshape(table.shape[0], -1, 2)
    return lax.bitcast_convert_type(t16, jnp.float32)


def kernel(atom_types, chemistry_types, emb_table, chem_table):
    a = atom_types.reshape(BL // CH, CH).astype(jnp.int32)
    c = chemistry_types.reshape(BL // CH, CH).astype(jnp.int32)
    out = _emb_lookup(a, c, _pack_table(emb_table), _pack_table(chem_table))
    o16 = lax.bitcast_convert_type(out[:, :PO], jnp.bfloat16)
    return o16.reshape(BL, D_OUT).astype(jnp.float32).reshape(B, L, D_OUT)


# speedup vs baseline: 3.1973x; 3.1973x over previous
import functools

import jax
import jax.numpy as jnp
from jax import lax
from jax.experimental import pallas as pl
from jax.experimental.pallas import tpu as pltpu
from jax.experimental.pallas import tpu_sc as plsc

B, L = 4096, 200
D_A, D_C = 64, 32
D_OUT = D_A + D_C
BL = B * L

NC, NS = 2, 16
NW = NC * NS
IG = 128
CH = 256
PER_W = BL // NW
NITER = PER_W // CH
NIDX = PER_W // IG
NBUF = 3
PF = 2
HEAD = 3
TAIL = 4
assert (NITER - HEAD - TAIL) % NBUF == 0 and PF < NBUF <= HEAD + (NBUF - PF)


def _emb_body(aidx_hbm, cidx_hbm, emb_hbm, chem_hbm, out_hbm,
              aidx_v, cidx_v, abuf, cbuf, gsems, wsems):
    wid = lax.axis_index("s") * NC + lax.axis_index("c")
    row0 = wid * PER_W
    it0 = wid * NIDX

    pltpu.sync_copy(aidx_hbm.at[pl.ds(it0, NIDX)], aidx_v)
    pltpu.sync_copy(cidx_hbm.at[pl.ds(it0, NIDX)], cidx_v)

    def gather_start(j, b):
        for h in range(CH // IG):
            pltpu.async_copy(emb_hbm.at[aidx_v.at[2 * j + h]],
                             abuf.at[b, pl.ds(h * IG, IG)], gsems.at[b])
            pltpu.async_copy(chem_hbm.at[cidx_v.at[2 * j + h]],
                             cbuf.at[b, pl.ds(h * IG, IG)], gsems.at[b])

    def gather_wait(b):
        for h in range(CH // IG):
            pltpu.make_async_copy(emb_hbm.at[aidx_v.at[0]],
                                  abuf.at[b, pl.ds(h * IG, IG)],
                                  gsems.at[b]).wait()
            pltpu.make_async_copy(chem_hbm.at[cidx_v.at[0]],
                                  cbuf.at[b, pl.ds(h * IG, IG)],
                                  gsems.at[b]).wait()

    def write_start(j, b):
        r = row0 + j * CH
        pltpu.async_copy(abuf.at[b], out_hbm.at[pl.ds(r, CH), pl.ds(0, D_A)],
                         wsems.at[b])
        pltpu.async_copy(cbuf.at[b], out_hbm.at[pl.ds(r, CH), pl.ds(D_A, D_C)],
                         wsems.at[b])

    def write_wait(b):
        pltpu.make_async_copy(abuf.at[b],
                              out_hbm.at[pl.ds(row0, CH), pl.ds(0, D_A)],
                              wsems.at[b]).wait()
        pltpu.make_async_copy(cbuf.at[b],
                              out_hbm.at[pl.ds(row0, CH), pl.ds(D_A, D_C)],
                              wsems.at[b]).wait()

    def step(j, b, bn, wait_w, prefetch):
        gather_wait(b)
        write_start(j, b)
        if prefetch:
            if wait_w:
                write_wait(bn)
            gather_start(j + PF, bn)

    for p in range(PF):
        gather_start(p, p % NBUF)

    for j in range(HEAD):
        step(j, j % NBUF, (j + PF) % NBUF, wait_w=(j >= NBUF - PF),
             prefetch=True)

    @pl.loop(HEAD, NITER - TAIL, step=NBUF)
    def _main(g):
        for b in range(NBUF):
            step(g + b, b, (b + PF) % NBUF, wait_w=True, prefetch=True)

    for j in range(NITER - TAIL, NITER):
        step(j, j % NBUF, (j + PF) % NBUF, wait_w=True,
             prefetch=(j + PF < NITER))

    for w in range(NITER - NBUF, NITER):
        write_wait(w % NBUF)


_emb_lookup = functools.partial(
    pl.kernel,
    out_type=jax.ShapeDtypeStruct((BL, 128), jnp.float32),
    mesh=plsc.VectorSubcoreMesh(core_axis_name="c", subcore_axis_name="s",
                                num_cores=NC, num_subcores=NS),
    scratch_types=[
        pltpu.VMEM((BL // NW // IG, IG), jnp.int32),
        pltpu.VMEM((BL // NW // IG, IG), jnp.int32),
        pltpu.VMEM((NBUF, CH, D_A), jnp.float32),
        pltpu.VMEM((NBUF, CH, D_C), jnp.float32),
        pltpu.SemaphoreType.DMA((NBUF,)),
        pltpu.SemaphoreType.DMA((NBUF,)),
    ],
    compiler_params=pltpu.CompilerParams(use_tc_tiling_on_sc=False),
)(_emb_body)


def kernel(atom_types, chemistry_types, emb_table, chem_table):
    a = atom_types.reshape(BL // IG, IG).astype(jnp.int32)
    c = chemistry_types.reshape(BL // IG, IG).astype(jnp.int32)
    out = _emb_lookup(a, c, emb_table, chem_table)
    return out[:, :D_OUT].reshape(B, L, D_OUT)


# speedup vs baseline: 5.4052x; 1.6905x over previous
import functools

import jax
import jax.numpy as jnp
from jax import lax
from jax.experimental import pallas as pl
from jax.experimental.pallas import tpu as pltpu
from jax.experimental.pallas import tpu_sc as plsc

B, L = 4096, 200
D_A, D_C = 64, 32
D_OUT = D_A + D_C
BL = B * L

NC, NS = 2, 16
NW = NC * NS
IG = 128
CH = 256
PER_W = BL // NW
NITER = PER_W // CH
NIDX = PER_W // IG
NBUF = 3
PF = 2
HEAD = 3
TAIL = 4
assert (NITER - HEAD - TAIL) % NBUF == 0 and PF < NBUF <= HEAD + (NBUF - PF)


def _emb_body(aidx_hbm, cidx_hbm, emb_hbm, chem_hbm, out_hbm,
              aidx_v, cidx_v, abuf, cbuf, emb_sp, chem_sp, gsems, wsems):
    sid = lax.axis_index("s")
    wid = sid * NC + lax.axis_index("c")
    row0 = wid * PER_W
    it0 = wid * NIDX

    @pl.when(sid == 0)
    def _stage_tables():
        pltpu.sync_copy(emb_hbm, emb_sp)
        pltpu.sync_copy(chem_hbm, chem_sp)

    pltpu.sync_copy(aidx_hbm.at[pl.ds(it0, NIDX)], aidx_v)
    pltpu.sync_copy(cidx_hbm.at[pl.ds(it0, NIDX)], cidx_v)
    plsc.subcore_barrier()

    def gather_start(j, b):
        for h in range(CH // IG):
            pltpu.async_copy(emb_sp.at[aidx_v.at[2 * j + h]],
                             abuf.at[b, pl.ds(h * IG, IG)], gsems.at[b])
            pltpu.async_copy(chem_sp.at[cidx_v.at[2 * j + h]],
                             cbuf.at[b, pl.ds(h * IG, IG)], gsems.at[b])

    def gather_wait(b):
        for h in range(CH // IG):
            pltpu.make_async_copy(emb_sp.at[aidx_v.at[0]],
                                  abuf.at[b, pl.ds(h * IG, IG)],
                                  gsems.at[b]).wait()
            pltpu.make_async_copy(chem_sp.at[cidx_v.at[0]],
                                  cbuf.at[b, pl.ds(h * IG, IG)],
                                  gsems.at[b]).wait()

    def write_start(j, b):
        r = row0 + j * CH
        pltpu.async_copy(abuf.at[b], out_hbm.at[pl.ds(r, CH), pl.ds(0, D_A)],
                         wsems.at[b])
        pltpu.async_copy(cbuf.at[b], out_hbm.at[pl.ds(r, CH), pl.ds(D_A, D_C)],
                         wsems.at[b])

    def write_wait(b):
        pltpu.make_async_copy(abuf.at[b],
                              out_hbm.at[pl.ds(row0, CH), pl.ds(0, D_A)],
                              wsems.at[b]).wait()
        pltpu.make_async_copy(cbuf.at[b],
                              out_hbm.at[pl.ds(row0, CH), pl.ds(D_A, D_C)],
                              wsems.at[b]).wait()

    def step(j, b, bn, wait_w, prefetch):
        gather_wait(b)
        write_start(j, b)
        if prefetch:
            if wait_w:
                write_wait(bn)
            gather_start(j + PF, bn)

    for p in range(PF):
        gather_start(p, p % NBUF)

    for j in range(HEAD):
        step(j, j % NBUF, (j + PF) % NBUF, wait_w=(j >= NBUF - PF),
             prefetch=True)

    @pl.loop(HEAD, NITER - TAIL, step=NBUF)
    def _main(g):
        for b in range(NBUF):
            step(g + b, b, (b + PF) % NBUF, wait_w=True, prefetch=True)

    for j in range(NITER - TAIL, NITER):
        step(j, j % NBUF, (j + PF) % NBUF, wait_w=True,
             prefetch=(j + PF < NITER))

    for w in range(NITER - NBUF, NITER):
        write_wait(w % NBUF)


_emb_lookup = functools.partial(
    pl.kernel,
    out_type=jax.ShapeDtypeStruct((BL, 128), jnp.float32),
    mesh=plsc.VectorSubcoreMesh(core_axis_name="c", subcore_axis_name="s",
                                num_cores=NC, num_subcores=NS),
    scratch_types=[
        pltpu.VMEM((BL // NW // IG, IG), jnp.int32),
        pltpu.VMEM((BL // NW // IG, IG), jnp.int32),
        pltpu.VMEM((NBUF, CH, D_A), jnp.float32),
        pltpu.VMEM((NBUF, CH, D_C), jnp.float32),
        pltpu.VMEM_SHARED((1000, D_A), jnp.float32),
        pltpu.VMEM_SHARED((1000, D_C), jnp.float32),
        pltpu.SemaphoreType.DMA((NBUF,)),
        pltpu.SemaphoreType.DMA((NBUF,)),
    ],
    compiler_params=pltpu.CompilerParams(use_tc_tiling_on_sc=False),
)(_emb_body)


def kernel(atom_types, chemistry_types, emb_table, chem_table):
    a = atom_types.reshape(BL // IG, IG).astype(jnp.int32)
    c = chemistry_types.reshape(BL // IG, IG).astype(jnp.int32)
    out = _emb_lookup(a, c, emb_table, chem_table)
    return out[:, :D_OUT].reshape(B, L, D_OUT)
